# baseline (speedup 1.0000x reference)
import functools

import jax
import jax.numpy as jnp
from jax import lax
from jax.experimental import pallas as pl
from jax.experimental.pallas import tpu as pltpu
from jax.experimental.pallas import tpu_sc as plsc

_D = 2048
_I = 5632
_E = 8
_K = 2
_T = 4096
_TK = _T * _K

_BM = 128
_CAP = _TK + _E * _BM
_NB = _CAP // _BM
_BN1 = 1408
_NT1 = _I // _BN1
_BN2 = 1024
_NT2 = _D // _BN2
_RT = 512


def _router_body(x_ref, rwt_ref, e_ref, w_ref):
    logits = jnp.dot(x_ref[...], rwt_ref[...], preferred_element_type=jnp.float32)
    lane = jax.lax.broadcasted_iota(jnp.int32, logits.shape, 1)
    neginf = jnp.float32(-jnp.inf)
    logits = jnp.where(lane < _E, logits, neginf)
    m1 = jnp.max(logits, axis=1, keepdims=True)
    e1 = jnp.min(jnp.where(logits == m1, lane, 127), axis=1, keepdims=True)
    logits2 = jnp.where(lane == e1, neginf, logits)
    m2 = jnp.max(logits2, axis=1, keepdims=True)
    e2 = jnp.min(jnp.where(logits2 == m2, lane, 127), axis=1, keepdims=True)
    d = m2 - m1
    ed = jnp.exp(d)
    w2 = ed / (1.0 + ed)
    w1 = 1.0 - w2
    e_ref[...] = jnp.where(lane == 0, e1, jnp.where(lane == 1, e2, 0))
    w_ref[...] = jnp.where(lane == 0, w1, jnp.where(lane == 1, w2, 0.0))


def _ffn1_body(be_ref, xs_ref, wg_ref, wu_ref, act_ref):
    x = xs_ref[...]
    dn = (((1,), (1,)), ((), ()))
    wg = wg_ref[0].astype(jnp.bfloat16)
    wu = wu_ref[0].astype(jnp.bfloat16)
    g = jax.lax.dot_general(x, wg, dn, preferred_element_type=jnp.float32)
    u = jax.lax.dot_general(x, wu, dn, preferred_element_type=jnp.float32)
    act_ref[...] = (g * jax.nn.sigmoid(g) * u).astype(act_ref.dtype)


def _ffn2_body(be_ref, act_ref, w2_ref, ws_ref, y_ref):
    dn = (((1,), (1,)), ((), ()))
    y = jax.lax.dot_general(
        act_ref[...], w2_ref[0].astype(jnp.bfloat16), dn,
        preferred_element_type=jnp.float32,
    )
    y_ref[...] = (y * ws_ref[:, 0:1]).astype(y_ref.dtype)


_NW = 32
_TPW = _T // _NW
_CH = 16


def _combine_body(y_hbm, pos0_hbm, pos1_hbm, out_hbm, i0_v, i1_v, r0_v, r1_v, o_v, sem):
    wid = lax.axis_index("s") * 2 + lax.axis_index("c")
    base = wid * _TPW
    pltpu.sync_copy(pos0_hbm.at[pl.ds(base, _TPW)], i0_v)
    pltpu.sync_copy(pos1_hbm.at[pl.ds(base, _TPW)], i1_v)

    def chunk(c, carry):
        pltpu.async_copy(y_hbm.at[i0_v.at[pl.ds(c * _CH, _CH)]], r0_v, sem).wait()
        pltpu.async_copy(y_hbm.at[i1_v.at[pl.ds(c * _CH, _CH)]], r1_v, sem).wait()

        def row(r, carry2):
            def vec(j, carry3):
                a = r0_v[r, pl.ds(j * 16, 16)]
                b = r1_v[r, pl.ds(j * 16, 16)]
                o_v[r, pl.ds(j * 16, 16)] = a + b
                return carry3

            lax.fori_loop(0, _D // 16, vec, 0, unroll=8)
            return carry2

        lax.fori_loop(0, _CH, row, 0)
        pltpu.sync_copy(o_v, out_hbm.at[pl.ds(base + c * _CH, _CH)])
        return carry

    lax.fori_loop(0, _TPW // _CH, chunk, 0)


def _sc_combine(y, pos0, pos1):
    mesh = plsc.VectorSubcoreMesh(core_axis_name="c", subcore_axis_name="s")
    f = functools.partial(
        pl.kernel,
        mesh=mesh,
        out_type=jax.ShapeDtypeStruct((_T, _D), jnp.float32),
        scratch_types=[
            pltpu.VMEM((_TPW,), jnp.int32),
            pltpu.VMEM((_TPW,), jnp.int32),
            pltpu.VMEM((_CH, _D), jnp.float32),
            pltpu.VMEM((_CH, _D), jnp.float32),
            pltpu.VMEM((_CH, _D), jnp.float32),
            pltpu.SemaphoreType.DMA,
        ],
    )(_combine_body)
    return f(y, pos0, pos1)


def kernel(x, router_w, w1, w2):
    b, s, d = x.shape
    xf = x.reshape(_T, _D)

    rwt = jnp.zeros((_D, 128), jnp.float32).at[:, :_E].set(router_w.T)
    e_out, w_out = pl.pallas_call(
        _router_body,
        grid=(_T // _RT,),
        in_specs=[
            pl.BlockSpec((_RT, _D), lambda i: (i, 0)),
            pl.BlockSpec((_D, 128), lambda i: (0, 0)),
        ],
        out_specs=[
            pl.BlockSpec((_RT, 128), lambda i: (i, 0)),
            pl.BlockSpec((_RT, 128), lambda i: (i, 0)),
        ],
        out_shape=[
            jax.ShapeDtypeStruct((_T, 128), jnp.int32),
            jax.ShapeDtypeStruct((_T, 128), jnp.float32),
        ],
    )(xf, rwt)
    e_tok = e_out[:, :_K]
    wt_tok = w_out[:, :_K]

    e_flat = e_tok.reshape(_TK)
    onehot = (e_flat[:, None] == jnp.arange(_E, dtype=jnp.int32)[None, :]).astype(
        jnp.int32
    )
    csum = jnp.cumsum(onehot, axis=0)
    counts = csum[-1]
    rank = jnp.sum((csum - onehot) * onehot, axis=1)
    padded = ((counts + _BM - 1) // _BM) * _BM
    pstart = jnp.concatenate([jnp.zeros((1,), jnp.int32), jnp.cumsum(padded)[:-1]])
    pos = pstart[e_flat] + rank
    row_src = jnp.zeros((_CAP,), jnp.int32).at[pos].set(
        jnp.arange(_TK, dtype=jnp.int32) // _K
    )
    block_expert = jnp.sum(
        (jnp.arange(_NB, dtype=jnp.int32)[None, :] * _BM) >= pstart[:, None], axis=0
    ).astype(jnp.int32) - 1

    ws_col = jnp.zeros((_CAP, 128), jnp.float32).at[pos].set(
        jnp.broadcast_to(wt_tok.reshape(_TK)[:, None], (_TK, 128))
    )

    x_sorted = jnp.take(xf, row_src, axis=0).astype(jnp.bfloat16)

    act = pl.pallas_call(
        _ffn1_body,
        grid_spec=pltpu.PrefetchScalarGridSpec(
            num_scalar_prefetch=1,
            grid=(_NT1, _NB),
            in_specs=[
                pl.BlockSpec((_BM, _D), lambda n, i, be: (i, 0)),
                pl.BlockSpec((1, _BN1, _D), lambda n, i, be: (be[i], n, 0)),
                pl.BlockSpec((1, _BN1, _D), lambda n, i, be: (be[i], n + _NT1, 0)),
            ],
            out_specs=pl.BlockSpec((_BM, _BN1), lambda n, i, be: (i, n)),
        ),
        out_shape=jax.ShapeDtypeStruct((_CAP, _I), jnp.bfloat16),
        )(block_expert, x_sorted, w1, w1)

    y = pl.pallas_call(
        _ffn2_body,
        grid_spec=pltpu.PrefetchScalarGridSpec(
            num_scalar_prefetch=1,
            grid=(_NT2, _NB),
            in_specs=[
                pl.BlockSpec((_BM, _I), lambda n, i, be: (i, 0)),
                pl.BlockSpec((1, _BN2, _I), lambda n, i, be: (be[i], n, 0)),
                pl.BlockSpec((_BM, 128), lambda n, i, be: (i, 0)),
            ],
            out_specs=pl.BlockSpec((_BM, _BN2), lambda n, i, be: (i, n)),
        ),
        out_shape=jax.ShapeDtypeStruct((_CAP, _D), jnp.float32),
        )(block_expert, act, w2, ws_col)

    pos2 = pos.reshape(_T, _K)
    out = _sc_combine(y, pos2[:, 0], pos2[:, 1])
    return out.reshape(b, s, d)


# speedup vs baseline: 1.6586x; 1.6586x over previous
import functools

import jax
import jax.numpy as jnp
from jax import lax
from jax.experimental import pallas as pl
from jax.experimental.pallas import tpu as pltpu
from jax.experimental.pallas import tpu_sc as plsc

_D = 2048
_I = 5632
_E = 8
_K = 2
_T = 4096
_TK = _T * _K

_BM = 256
_CAP = _TK + _E * _BM
_NB = _CAP // _BM
_BN1 = 1408
_NT1 = _I // _BN1
_BN2 = 1024
_NT2 = _D // _BN2
_RT = 512


def _router_body(x_ref, rwt_ref, e_ref, w_ref):
    logits = jnp.dot(x_ref[...], rwt_ref[...], preferred_element_type=jnp.float32)
    lane = jax.lax.broadcasted_iota(jnp.int32, logits.shape, 1)
    neginf = jnp.float32(-jnp.inf)
    logits = jnp.where(lane < _E, logits, neginf)
    m1 = jnp.max(logits, axis=1, keepdims=True)
    e1 = jnp.min(jnp.where(logits == m1, lane, 127), axis=1, keepdims=True)
    logits2 = jnp.where(lane == e1, neginf, logits)
    m2 = jnp.max(logits2, axis=1, keepdims=True)
    e2 = jnp.min(jnp.where(logits2 == m2, lane, 127), axis=1, keepdims=True)
    d = m2 - m1
    ed = jnp.exp(d)
    w2 = ed / (1.0 + ed)
    w1 = 1.0 - w2
    e_ref[...] = jnp.where(lane == 0, e1, jnp.where(lane == 1, e2, 0))
    w_ref[...] = jnp.where(lane == 0, w1, jnp.where(lane == 1, w2, 0.0))


def _ffn1_body(be_ref, xs_ref, wg_ref, wu_ref, act_ref):
    x = xs_ref[...]
    dn = (((1,), (1,)), ((), ()))
    wg = wg_ref[0].astype(jnp.bfloat16)
    wu = wu_ref[0].astype(jnp.bfloat16)
    g = jax.lax.dot_general(x, wg, dn, preferred_element_type=jnp.float32)
    u = jax.lax.dot_general(x, wu, dn, preferred_element_type=jnp.float32)
    act_ref[...] = (g * jax.nn.sigmoid(g) * u).astype(act_ref.dtype)


def _ffn2_body(be_ref, act_ref, w2_ref, ws_ref, y_ref):
    dn = (((1,), (1,)), ((), ()))
    y = jax.lax.dot_general(
        act_ref[...], w2_ref[0].astype(jnp.bfloat16), dn,
        preferred_element_type=jnp.float32,
    )
    y_ref[...] = (y * ws_ref[:, 0:1]).astype(y_ref.dtype)


_NW = 32
_TPW = _T // _NW
_CH = 8
_NCH = _TPW // _CH


def _combine_body(y_hbm, pos0_hbm, pos1_hbm, out_hbm,
                  i0_v, i1_v, ra0, ra1, rb0, rb1, sa, sb):
    wid = lax.axis_index("s") * 2 + lax.axis_index("c")
    base = wid * _TPW
    pltpu.sync_copy(pos0_hbm.at[pl.ds(base, _TPW)], i0_v)
    pltpu.sync_copy(pos1_hbm.at[pl.ds(base, _TPW)], i1_v)
    bufs = [(ra0, ra1, sa), (rb0, rb1, sb)]

    def start(c, b):
        b0, b1, sem = bufs[b]
        c0 = pltpu.async_copy(y_hbm.at[i0_v.at[pl.ds(c * _CH, _CH)]], b0, sem)
        c1 = pltpu.async_copy(y_hbm.at[i1_v.at[pl.ds(c * _CH, _CH)]], b1, sem)
        return (c0, c1)

    pending = start(0, 0)
    for c in range(_NCH):
        b0, b1, _ = bufs[c % 2]
        cur = pending
        if c + 1 < _NCH:
            pending = start(c + 1, (c + 1) % 2)
        cur[0].wait()
        cur[1].wait()

        def row(r, carry2):
            def vec(j, carry3):
                b0[r, pl.ds(j * 16, 16)] = (
                    b0[r, pl.ds(j * 16, 16)] + b1[r, pl.ds(j * 16, 16)]
                )
                return carry3

            lax.fori_loop(0, _D // 16, vec, 0, unroll=8)
            return carry2

        lax.fori_loop(0, _CH, row, 0)
        pltpu.sync_copy(b0, out_hbm.at[pl.ds(base + c * _CH, _CH)])


def _sc_combine(y, pos0, pos1):
    mesh = plsc.VectorSubcoreMesh(core_axis_name="c", subcore_axis_name="s")
    f = functools.partial(
        pl.kernel,
        mesh=mesh,
        out_type=jax.ShapeDtypeStruct((_T, _D), jnp.float32),
        scratch_types=[
            pltpu.VMEM((_TPW,), jnp.int32),
            pltpu.VMEM((_TPW,), jnp.int32),
            pltpu.VMEM((_CH, _D), jnp.float32),
            pltpu.VMEM((_CH, _D), jnp.float32),
            pltpu.VMEM((_CH, _D), jnp.float32),
            pltpu.VMEM((_CH, _D), jnp.float32),
            pltpu.SemaphoreType.DMA,
            pltpu.SemaphoreType.DMA,
        ],
    )(_combine_body)
    return f(y, pos0, pos1)


def kernel(x, router_w, w1, w2):
    b, s, d = x.shape
    xf = x.reshape(_T, _D)

    rwt = jnp.zeros((_D, 128), jnp.float32).at[:, :_E].set(router_w.T)
    e_out, w_out = pl.pallas_call(
        _router_body,
        grid=(_T // _RT,),
        in_specs=[
            pl.BlockSpec((_RT, _D), lambda i: (i, 0)),
            pl.BlockSpec((_D, 128), lambda i: (0, 0)),
        ],
        out_specs=[
            pl.BlockSpec((_RT, 128), lambda i: (i, 0)),
            pl.BlockSpec((_RT, 128), lambda i: (i, 0)),
        ],
        out_shape=[
            jax.ShapeDtypeStruct((_T, 128), jnp.int32),
            jax.ShapeDtypeStruct((_T, 128), jnp.float32),
        ],
    )(xf, rwt)
    e_tok = e_out[:, :_K]
    wt_tok = w_out[:, :_K]

    e_flat = e_tok.reshape(_TK)
    onehot = (e_flat[:, None] == jnp.arange(_E, dtype=jnp.int32)[None, :]).astype(
        jnp.int32
    )
    csum = jnp.cumsum(onehot, axis=0)
    counts = csum[-1]
    rank = jnp.sum((csum - onehot) * onehot, axis=1)
    padded = ((counts + _BM - 1) // _BM) * _BM
    pstart = jnp.concatenate([jnp.zeros((1,), jnp.int32), jnp.cumsum(padded)[:-1]])
    pos = pstart[e_flat] + rank
    row_src = jnp.zeros((_CAP,), jnp.int32).at[pos].set(
        jnp.arange(_TK, dtype=jnp.int32) // _K
    )
    block_expert = jnp.sum(
        (jnp.arange(_NB, dtype=jnp.int32)[None, :] * _BM) >= pstart[:, None], axis=0
    ).astype(jnp.int32) - 1

    ws_col = jnp.zeros((_CAP, 128), jnp.float32).at[pos].set(
        jnp.broadcast_to(wt_tok.reshape(_TK)[:, None], (_TK, 128))
    )

    x_sorted = jnp.take(xf, row_src, axis=0).astype(jnp.bfloat16)

    act = pl.pallas_call(
        _ffn1_body,
        grid_spec=pltpu.PrefetchScalarGridSpec(
            num_scalar_prefetch=1,
            grid=(_NT1, _NB),
            in_specs=[
                pl.BlockSpec((_BM, _D), lambda n, i, be: (i, 0)),
                pl.BlockSpec((1, _BN1, _D), lambda n, i, be: (be[i], n, 0)),
                pl.BlockSpec((1, _BN1, _D), lambda n, i, be: (be[i], n + _NT1, 0)),
            ],
            out_specs=pl.BlockSpec((_BM, _BN1), lambda n, i, be: (i, n)),
        ),
        out_shape=jax.ShapeDtypeStruct((_CAP, _I), jnp.bfloat16),
    )(block_expert, x_sorted, w1, w1)

    y = pl.pallas_call(
        _ffn2_body,
        grid_spec=pltpu.PrefetchScalarGridSpec(
            num_scalar_prefetch=1,
            grid=(_NT2, _NB),
            in_specs=[
                pl.BlockSpec((_BM, _I), lambda n, i, be: (i, 0)),
                pl.BlockSpec((1, _BN2, _I), lambda n, i, be: (be[i], n, 0)),
                pl.BlockSpec((_BM, 128), lambda n, i, be: (i, 0)),
            ],
            out_specs=pl.BlockSpec((_BM, _BN2), lambda n, i, be: (i, n)),
        ),
        out_shape=jax.ShapeDtypeStruct((_CAP, _D), jnp.float32),
    )(block_expert, act, w2, ws_col)

    pos2 = pos.reshape(_T, _K)
    out = _sc_combine(y, pos2[:, 0], pos2[:, 1])
    return out.reshape(b, s, d)
